# Initial kernel scaffold; baseline (speedup 1.0000x reference)
import numpy as np
import jax
import jax.numpy as jnp
from jax import lax
from jax.experimental import pallas as pl
from jax.experimental.pallas import tpu as pltpu
from jax.experimental.pallas import tpu_sc as plsc

_N, _E, _D, _H, _G, _C = 10000, 320000, 128, 128, 64, 10
_BN_SCALE = float(1.0 / np.sqrt(1.0 + 1e-5))

_NC, _NS = 2, 16
_NW = _NC * _NS
_CH = 128
_NCHUNK = _E // _CH
_CPT = _NCHUNK // _NW
_CREM = _NCHUNK - _CPT * _NW
_RPT = _N // _NS
_ZR = 125

_R = 2000
_NB = _N // _R


def _agg_body(x_hbm, src_hbm, dst_hbm, out_hbm, srcv, dstv, rows_v, zer_v,
              acc_sh, sem):
    c = lax.axis_index("c")
    s = lax.axis_index("s")
    w = s * _NC + c

    zeros16 = jnp.zeros((16,), jnp.float32)

    def zfill(i, carry):
        r = i // (_D // 16)
        col = (i % (_D // 16)) * 16
        zer_v[r, pl.ds(col, 16)] = zeros16
        return carry

    lax.fori_loop(0, _ZR * (_D // 16), zfill, 0)
    r0 = s * _RPT
    for t in range(_RPT // _ZR):
        pltpu.sync_copy(zer_v, acc_sh.at[pl.ds(r0 + t * _ZR, _ZR), :])
    plsc.subcore_barrier()

    base = w * _CPT + jnp.minimum(w, _CREM)
    cnt = _CPT + (w < _CREM).astype(jnp.int32)

    def body(k, carry):
        cb = base + k
        pltpu.sync_copy(src_hbm.at[cb], srcv)
        pltpu.sync_copy(dst_hbm.at[cb], dstv)
        pltpu.async_copy(x_hbm.at[srcv], rows_v, sem).wait()
        pltpu.sync_copy(rows_v, acc_sh.at[dstv], add=True)
        return carry

    lax.fori_loop(0, cnt, body, 0)

    plsc.subcore_barrier()
    pltpu.sync_copy(acc_sh.at[pl.ds(r0, _RPT), :],
                    out_hbm.at[c, pl.ds(r0, _RPT), :])


_agg = pl.kernel(
    _agg_body,
    out_type=jax.ShapeDtypeStruct((_NC, _N, _D), jnp.float32),
    mesh=plsc.VectorSubcoreMesh(core_axis_name="c", subcore_axis_name="s"),
    scratch_types=[
        pltpu.VMEM((_CH,), jnp.int32),
        pltpu.VMEM((_CH,), jnp.int32),
        pltpu.VMEM((_CH, _D), jnp.float32),
        pltpu.VMEM((_ZR, _D), jnp.float32),
        pltpu.VMEM_SHARED((_N, _D), jnp.float32),
        pltpu.SemaphoreType.DMA,
    ],
)


def _mlp2_body(x_ref, a_ref, w1_ref, b1_ref, w2_ref, b2_ref, g_ref, be_ref,
               bat_ref, h_ref, p_ref):
    i = pl.program_id(0)
    h = x_ref[...] + a_ref[0] + a_ref[1]
    t = jnp.maximum(
        jnp.dot(h, w1_ref[...], preferred_element_type=jnp.float32)
        + b1_ref[...], 0.0)
    u = (jnp.dot(t, w2_ref[...], preferred_element_type=jnp.float32)
         + b2_ref[...])
    hn = jnp.maximum(u * _BN_SCALE * g_ref[...] + be_ref[...], 0.0)
    h_ref[...] = hn
    bat = bat_ref[0, 0, :]
    oh = (bat[:, None] == lax.broadcasted_iota(jnp.int32, (_R, _G), 1)
          ).astype(jnp.float32)
    p = lax.dot_general(oh, hn, (((0,), (0,)), ((), ())),
                        preferred_element_type=jnp.float32)

    @pl.when(i == 0)
    def _():
        p_ref[...] = p

    @pl.when(i != 0)
    def _():
        p_ref[...] = p_ref[...] + p


def _mlp3_body(x_ref, a_ref, w_ref, b_ref, g_ref, be_ref, bat_ref, p_ref):
    i = pl.program_id(0)
    h = x_ref[...] + a_ref[0] + a_ref[1]
    t = jnp.maximum(
        jnp.dot(h, w_ref[...], preferred_element_type=jnp.float32)
        + b_ref[...], 0.0)
    hn = jnp.maximum(t * _BN_SCALE * g_ref[...] + be_ref[...], 0.0)
    bat = bat_ref[0, 0, :]
    oh = (bat[:, None] == lax.broadcasted_iota(jnp.int32, (_R, _G), 1)
          ).astype(jnp.float32)
    p = lax.dot_general(oh, hn, (((0,), (0,)), ((), ())),
                        preferred_element_type=jnp.float32)

    @pl.when(i == 0)
    def _():
        p_ref[...] = p

    @pl.when(i != 0)
    def _():
        p_ref[...] = p_ref[...] + p


def _readout_body(p1_ref, p2_ref, p3_ref, wa_ref, wb_ref, wc_ref, b1_ref,
                  w2_ref, b2_ref, out_ref):
    z = (jnp.dot(p1_ref[...], wa_ref[...], preferred_element_type=jnp.float32)
         + jnp.dot(p2_ref[...], wb_ref[...], preferred_element_type=jnp.float32)
         + jnp.dot(p3_ref[...], wc_ref[...], preferred_element_type=jnp.float32)
         + b1_ref[...])
    z = jnp.maximum(z, 0.0)
    out_ref[...] = (jnp.dot(z, w2_ref[...], preferred_element_type=jnp.float32)
                    + b2_ref[...])


def _full_spec(shape):
    nd = len(shape)
    return pl.BlockSpec(shape, lambda i=0, _n=nd: (0,) * _n)


def _mlp2_call(x, a, w1, b1, w2, b2, g, be, bat3, dh):
    return pl.pallas_call(
        _mlp2_body,
        grid=(_NB,),
        in_specs=[
            pl.BlockSpec((_R, _D), lambda i: (i, 0)),
            pl.BlockSpec((_NC, _R, _D), lambda i: (0, i, 0)),
            _full_spec((_D, dh)),
            _full_spec((1, dh)),
            _full_spec((dh, dh)),
            _full_spec((1, dh)),
            _full_spec((1, dh)),
            _full_spec((1, dh)),
            pl.BlockSpec((1, 1, _R), lambda i: (i, 0, 0)),
        ],
        out_specs=[
            pl.BlockSpec((_R, dh), lambda i: (i, 0)),
            pl.BlockSpec((_G, dh), lambda i: (0, 0)),
        ],
        out_shape=[
            jax.ShapeDtypeStruct((_N, dh), jnp.float32),
            jax.ShapeDtypeStruct((_G, dh), jnp.float32),
        ],
    )(x, a, w1, b1, w2, b2, g, be, bat3)


def _mlp3_call(x, a, w, b, g, be, bat3, dh):
    return pl.pallas_call(
        _mlp3_body,
        grid=(_NB,),
        in_specs=[
            pl.BlockSpec((_R, _D), lambda i: (i, 0)),
            pl.BlockSpec((_NC, _R, _D), lambda i: (0, i, 0)),
            _full_spec((_D, dh)),
            _full_spec((1, dh)),
            _full_spec((1, dh)),
            _full_spec((1, dh)),
            pl.BlockSpec((1, 1, _R), lambda i: (i, 0, 0)),
        ],
        out_specs=pl.BlockSpec((_G, dh), lambda i: (0, 0)),
        out_shape=jax.ShapeDtypeStruct((_G, dh), jnp.float32),
    )(x, a, w, b, g, be, bat3)


def _readout_call(p1, p2, p3, wa, wb, wc, b1, w2, b2):
    return pl.pallas_call(
        _readout_body,
        in_specs=[_full_spec(t.shape) for t in
                  (p1, p2, p3, wa, wb, wc, b1, w2, b2)],
        out_specs=_full_spec((_G, _C)),
        out_shape=jax.ShapeDtypeStruct((_G, _C), jnp.float32),
    )(p1, p2, p3, wa, wb, wc, b1, w2, b2)


def kernel(x, edge_index, batch, W1a, b1a, W1b, b1b, W2a, b2a, W2b, b2b, W3,
           b3, g1, be1, g2, be2, g3, be3, Wl1, bl1, Wl2, bl2):
    src = edge_index[0].reshape(_NCHUNK, _CH)
    dst = edge_index[1].reshape(_NCHUNK, _CH)
    bat3 = batch.reshape(_NB, 1, _R)

    r = lambda v: v.reshape(1, -1)

    a = _agg(x, src, dst)
    h1, p1 = _mlp2_call(x, a, W1a, r(b1a), W1b, r(b1b), r(g1), r(be1), bat3,
                        _H)
    a = _agg(h1, src, dst)
    h2, p2 = _mlp2_call(h1, a, W2a, r(b2a), W2b, r(b2b), r(g2), r(be2), bat3,
                        _H)
    a = _agg(h2, src, dst)
    p3 = _mlp3_call(h2, a, W3, r(b3), r(g3), r(be3), bat3, 512)

    return _readout_call(p1, p2, p3, Wl1[:_H], Wl1[_H:2 * _H], Wl1[2 * _H:],
                         r(bl1), Wl2, r(bl2))


# speedup vs baseline: 8.0597x; 8.0597x over previous
import numpy as np
import jax
import jax.numpy as jnp
from jax import lax
from jax.experimental import pallas as pl
from jax.experimental.pallas import tpu as pltpu
from jax.experimental.pallas import tpu_sc as plsc

_N, _E, _D, _H, _G, _C = 10000, 320000, 128, 128, 64, 10
_BN_SCALE = float(1.0 / np.sqrt(1.0 + 1e-5))

_NC, _NS = 2, 16
_NW = _NC * _NS
_CH = 128
_SB = 2
_NSB = _E // (_CH * _SB)
_SPT = _NSB // _NW
_SREM = _NSB - _SPT * _NW
_NPAD = 10112
_RPT = _NPAD // _NS

_R = 2000
_NB = _N // _R


def _agg_body(x_hbm, src_hbm, dst_hbm, zinit_hbm, out_hbm, srcv, dstv, rows_v,
              acc_sh, sem, sem2):
    c = lax.axis_index("c")
    s = lax.axis_index("s")
    w = s * _NC + c

    r0 = s * _RPT
    pltpu.sync_copy(zinit_hbm.at[pl.ds(r0, _RPT), :],
                    acc_sh.at[pl.ds(r0, _RPT), :])
    plsc.subcore_barrier()

    base = w * _SPT + jnp.minimum(w, _SREM)
    cnt = _SPT + (w < _SREM).astype(jnp.int32)

    def body(k, carry):
        sb = base + k
        pltpu.sync_copy(src_hbm.at[sb], srcv)
        pltpu.sync_copy(dst_hbm.at[sb], dstv)
        gathers = [
            pltpu.async_copy(x_hbm.at[srcv.at[j]], rows_v.at[j], sem)
            for j in range(_SB)
        ]
        for g in gathers:
            g.wait()
        scatters = [
            pltpu.async_copy(rows_v.at[j], acc_sh.at[dstv.at[j]], sem2,
                             add=True)
            for j in range(_SB)
        ]
        for sc_ in scatters:
            sc_.wait()
        return carry

    lax.fori_loop(0, cnt, body, 0)

    plsc.subcore_barrier()
    pltpu.sync_copy(acc_sh.at[pl.ds(r0, _RPT), :],
                    out_hbm.at[c, pl.ds(r0, _RPT), :])


_agg_kernel_cache = {}


def _agg_run(x, src, dst, zinit):
    if "k" not in _agg_kernel_cache:
        _agg_kernel_cache["k"] = pl.kernel(
            _agg_body,
            out_type=jax.ShapeDtypeStruct((_NC, _NPAD, _D), jnp.float32),
            mesh=plsc.VectorSubcoreMesh(core_axis_name="c",
                                        subcore_axis_name="s",
                                        num_cores=_NC, num_subcores=_NS),
            scratch_types=[
                pltpu.VMEM((_SB, _CH), jnp.int32),
                pltpu.VMEM((_SB, _CH), jnp.int32),
                pltpu.VMEM((_SB, _CH, _D), jnp.float32),
                pltpu.VMEM_SHARED((_NPAD, _D), jnp.float32),
                pltpu.SemaphoreType.DMA,
                pltpu.SemaphoreType.DMA,
            ],
        )
    return _agg_kernel_cache["k"](x, src, dst, zinit)


def _mlp2_body(x_ref, a_ref, w1_ref, b1_ref, w2_ref, b2_ref, g_ref, be_ref,
               bat_ref, h_ref, p_ref):
    i = pl.program_id(0)
    h = x_ref[...] + a_ref[0] + a_ref[1]
    t = jnp.maximum(
        jnp.dot(h, w1_ref[...], preferred_element_type=jnp.float32)
        + b1_ref[...], 0.0)
    u = (jnp.dot(t, w2_ref[...], preferred_element_type=jnp.float32)
         + b2_ref[...])
    hn = jnp.maximum(u * _BN_SCALE * g_ref[...] + be_ref[...], 0.0)
    h_ref[...] = hn
    bat = bat_ref[0, 0, :]
    oh = (bat[:, None] == lax.broadcasted_iota(jnp.int32, (_R, _G), 1)
          ).astype(jnp.float32)
    p = lax.dot_general(oh, hn, (((0,), (0,)), ((), ())),
                        preferred_element_type=jnp.float32)

    @pl.when(i == 0)
    def _():
        p_ref[...] = p

    @pl.when(i != 0)
    def _():
        p_ref[...] = p_ref[...] + p


def _mlp3_body(x_ref, a_ref, w_ref, b_ref, g_ref, be_ref, bat_ref, p_ref):
    i = pl.program_id(0)
    h = x_ref[...] + a_ref[0] + a_ref[1]
    t = jnp.maximum(
        jnp.dot(h, w_ref[...], preferred_element_type=jnp.float32)
        + b_ref[...], 0.0)
    hn = jnp.maximum(t * _BN_SCALE * g_ref[...] + be_ref[...], 0.0)
    bat = bat_ref[0, 0, :]
    oh = (bat[:, None] == lax.broadcasted_iota(jnp.int32, (_R, _G), 1)
          ).astype(jnp.float32)
    p = lax.dot_general(oh, hn, (((0,), (0,)), ((), ())),
                        preferred_element_type=jnp.float32)

    @pl.when(i == 0)
    def _():
        p_ref[...] = p

    @pl.when(i != 0)
    def _():
        p_ref[...] = p_ref[...] + p


def _readout_body(p1_ref, p2_ref, p3_ref, wa_ref, wb_ref, wc_ref, b1_ref,
                  w2_ref, b2_ref, out_ref):
    z = (jnp.dot(p1_ref[...], wa_ref[...], preferred_element_type=jnp.float32)
         + jnp.dot(p2_ref[...], wb_ref[...], preferred_element_type=jnp.float32)
         + jnp.dot(p3_ref[...], wc_ref[...], preferred_element_type=jnp.float32)
         + b1_ref[...])
    z = jnp.maximum(z, 0.0)
    out_ref[...] = (jnp.dot(z, w2_ref[...], preferred_element_type=jnp.float32)
                    + b2_ref[...])


def _full_spec(shape):
    nd = len(shape)
    return pl.BlockSpec(shape, lambda i=0, _n=nd: (0,) * _n)


def _mlp2_call(x, a, w1, b1, w2, b2, g, be, bat3, dh):
    return pl.pallas_call(
        _mlp2_body,
        grid=(_NB,),
        in_specs=[
            pl.BlockSpec((_R, _D), lambda i: (i, 0)),
            pl.BlockSpec((_NC, _R, _D), lambda i: (0, i, 0)),
            _full_spec((_D, dh)),
            _full_spec((1, dh)),
            _full_spec((dh, dh)),
            _full_spec((1, dh)),
            _full_spec((1, dh)),
            _full_spec((1, dh)),
            pl.BlockSpec((1, 1, _R), lambda i: (i, 0, 0)),
        ],
        out_specs=[
            pl.BlockSpec((_R, dh), lambda i: (i, 0)),
            pl.BlockSpec((_G, dh), lambda i: (0, 0)),
        ],
        out_shape=[
            jax.ShapeDtypeStruct((_N, dh), jnp.float32),
            jax.ShapeDtypeStruct((_G, dh), jnp.float32),
        ],
    )(x, a, w1, b1, w2, b2, g, be, bat3)


def _mlp3_call(x, a, w, b, g, be, bat3, dh):
    return pl.pallas_call(
        _mlp3_body,
        grid=(_NB,),
        in_specs=[
            pl.BlockSpec((_R, _D), lambda i: (i, 0)),
            pl.BlockSpec((_NC, _R, _D), lambda i: (0, i, 0)),
            _full_spec((_D, dh)),
            _full_spec((1, dh)),
            _full_spec((1, dh)),
            _full_spec((1, dh)),
            pl.BlockSpec((1, 1, _R), lambda i: (i, 0, 0)),
        ],
        out_specs=pl.BlockSpec((_G, dh), lambda i: (0, 0)),
        out_shape=jax.ShapeDtypeStruct((_G, dh), jnp.float32),
    )(x, a, w, b, g, be, bat3)


def _readout_call(p1, p2, p3, wa, wb, wc, b1, w2, b2):
    return pl.pallas_call(
        _readout_body,
        in_specs=[_full_spec(t.shape) for t in
                  (p1, p2, p3, wa, wb, wc, b1, w2, b2)],
        out_specs=_full_spec((_G, _C)),
        out_shape=jax.ShapeDtypeStruct((_G, _C), jnp.float32),
    )(p1, p2, p3, wa, wb, wc, b1, w2, b2)


def kernel(x, edge_index, batch, W1a, b1a, W1b, b1b, W2a, b2a, W2b, b2b, W3,
           b3, g1, be1, g2, be2, g3, be3, Wl1, bl1, Wl2, bl2):
    src = edge_index[0].reshape(_NSB, _SB, _CH)
    dst = edge_index[1].reshape(_NSB, _SB, _CH)
    bat3 = batch.reshape(_NB, 1, _R)
    zinit = jnp.zeros((_NPAD, _D), jnp.float32)

    r = lambda v: v.reshape(1, -1)

    a = _agg_run(x, src, dst, zinit)
    h1, p1 = _mlp2_call(x, a, W1a, r(b1a), W1b, r(b1b), r(g1), r(be1), bat3,
                        _H)
    a = _agg_run(h1, src, dst, zinit)
    h2, p2 = _mlp2_call(h1, a, W2a, r(b2a), W2b, r(b2b), r(g2), r(be2), bat3,
                        _H)
    a = _agg_run(h2, src, dst, zinit)
    p3 = _mlp3_call(h2, a, W3, r(b3), r(g3), r(be3), bat3, 512)

    return _readout_call(p1, p2, p3, Wl1[:_H], Wl1[_H:2 * _H], Wl1[2 * _H:],
                         r(bl1), Wl2, r(bl2))


# speedup vs baseline: 9.9288x; 1.2319x over previous
import numpy as np
import jax
import jax.numpy as jnp
from jax import lax
from jax.experimental import pallas as pl
from jax.experimental.pallas import tpu as pltpu
from jax.experimental.pallas import tpu_sc as plsc

_N, _E, _D, _H, _G, _C = 10000, 320000, 128, 128, 64, 10
_BN_SCALE = float(1.0 / np.sqrt(1.0 + 1e-5))

_NC, _NS = 2, 16
_NW = _NC * _NS
_CH = 128
_SB = 4
_NSB = _E // (_CH * _SB)
_SPT = _NSB // _NW
_SREM = _NSB - _SPT * _NW
_NPAD = 10112
_RPT = _NPAD // _NS

_R = 2000
_NB = _N // _R


def _agg_body(x_hbm, src_hbm, dst_hbm, zinit_hbm, out_hbm, srcv, dstv, rows_v,
              acc_sh, gsA, gsB, ssA, ssB):
    c = lax.axis_index("c")
    s = lax.axis_index("s")
    w = s * _NC + c

    r0 = s * _RPT
    pltpu.sync_copy(zinit_hbm.at[pl.ds(r0, _RPT), :],
                    acc_sh.at[pl.ds(r0, _RPT), :])
    plsc.subcore_barrier()

    base = w * _SPT + jnp.minimum(w, _SREM)
    cnt = _SPT + (w < _SREM).astype(jnp.int32)
    gsems = (gsA, gsB)
    ssems = (ssA, ssB)
    dummy = x_hbm.at[pl.ds(0, _CH), :]

    def idx_load(q, pq):
        pltpu.sync_copy(src_hbm.at[q], srcv.at[pq])
        pltpu.sync_copy(dst_hbm.at[q], dstv.at[pq])

    def g_fire(pq, j, b):
        pltpu.async_copy(x_hbm.at[srcv.at[pq, j]], rows_v.at[b], gsems[b])

    def s_fire(pq, j, b):
        pltpu.async_copy(rows_v.at[b], acc_sh.at[dstv.at[pq, j]], ssems[b],
                         add=True)

    def g_drain(b):
        pltpu.make_async_copy(dummy, rows_v.at[b], gsems[b]).wait()

    def s_drain(b):
        pltpu.make_async_copy(dummy, rows_v.at[b], ssems[b]).wait()

    idx_load(base, 0)
    g_fire(0, 0, 0)
    g_fire(0, 1, 1)
    g_drain(0); s_fire(0, 0, 0)
    g_drain(1); s_fire(0, 1, 1)
    s_drain(0); g_fire(0, 2, 0)
    s_drain(1); g_fire(0, 3, 1)

    def body(i, carry):
        q = base + i
        p = lax.rem(i, 2)
        pp = 1 - p
        idx_load(q, p)
        g_drain(0); s_fire(pp, 2, 0)
        g_drain(1); s_fire(pp, 3, 1)
        s_drain(0); g_fire(p, 0, 0)
        s_drain(1); g_fire(p, 1, 1)
        g_drain(0); s_fire(p, 0, 0)
        g_drain(1); s_fire(p, 1, 1)
        s_drain(0); g_fire(p, 2, 0)
        s_drain(1); g_fire(p, 3, 1)
        return carry

    lax.fori_loop(1, cnt, body, 0)

    pl_ = lax.rem(cnt - 1, 2)
    g_drain(0); s_fire(pl_, 2, 0)
    g_drain(1); s_fire(pl_, 3, 1)
    s_drain(0)
    s_drain(1)

    plsc.subcore_barrier()
    pltpu.sync_copy(acc_sh.at[pl.ds(r0, _RPT), :],
                    out_hbm.at[c, pl.ds(r0, _RPT), :])


_agg_kernel_cache = {}


def _agg_run(x, src, dst, zinit):
    if "k" not in _agg_kernel_cache:
        _agg_kernel_cache["k"] = pl.kernel(
            _agg_body,
            out_type=jax.ShapeDtypeStruct((_NC, _NPAD, _D), jnp.float32),
            mesh=plsc.VectorSubcoreMesh(core_axis_name="c",
                                        subcore_axis_name="s",
                                        num_cores=_NC, num_subcores=_NS),
            scratch_types=[
                pltpu.VMEM((2, _SB, _CH), jnp.int32),
                pltpu.VMEM((2, _SB, _CH), jnp.int32),
                pltpu.VMEM((2, _CH, _D), jnp.float32),
                pltpu.VMEM_SHARED((_NPAD, _D), jnp.float32),
                pltpu.SemaphoreType.DMA,
                pltpu.SemaphoreType.DMA,
                pltpu.SemaphoreType.DMA,
                pltpu.SemaphoreType.DMA,
            ],
        )
    return _agg_kernel_cache["k"](x, src, dst, zinit)


def _mlp2_body(x_ref, a_ref, w1_ref, b1_ref, w2_ref, b2_ref, g_ref, be_ref,
               bat_ref, h_ref, p_ref):
    i = pl.program_id(0)
    h = x_ref[...] + a_ref[0] + a_ref[1]
    t = jnp.maximum(
        jnp.dot(h, w1_ref[...], preferred_element_type=jnp.float32)
        + b1_ref[...], 0.0)
    u = (jnp.dot(t, w2_ref[...], preferred_element_type=jnp.float32)
         + b2_ref[...])
    hn = jnp.maximum(u * _BN_SCALE * g_ref[...] + be_ref[...], 0.0)
    h_ref[...] = hn
    bat = bat_ref[0, 0, :]
    oh = (bat[:, None] == lax.broadcasted_iota(jnp.int32, (_R, _G), 1)
          ).astype(jnp.float32)
    p = lax.dot_general(oh, hn, (((0,), (0,)), ((), ())),
                        preferred_element_type=jnp.float32)

    @pl.when(i == 0)
    def _():
        p_ref[...] = p

    @pl.when(i != 0)
    def _():
        p_ref[...] = p_ref[...] + p


def _mlp3_body(x_ref, a_ref, w_ref, b_ref, g_ref, be_ref, bat_ref, p_ref):
    i = pl.program_id(0)
    h = x_ref[...] + a_ref[0] + a_ref[1]
    t = jnp.maximum(
        jnp.dot(h, w_ref[...], preferred_element_type=jnp.float32)
        + b_ref[...], 0.0)
    hn = jnp.maximum(t * _BN_SCALE * g_ref[...] + be_ref[...], 0.0)
    bat = bat_ref[0, 0, :]
    oh = (bat[:, None] == lax.broadcasted_iota(jnp.int32, (_R, _G), 1)
          ).astype(jnp.float32)
    p = lax.dot_general(oh, hn, (((0,), (0,)), ((), ())),
                        preferred_element_type=jnp.float32)

    @pl.when(i == 0)
    def _():
        p_ref[...] = p

    @pl.when(i != 0)
    def _():
        p_ref[...] = p_ref[...] + p


def _readout_body(p1_ref, p2_ref, p3_ref, wa_ref, wb_ref, wc_ref, b1_ref,
                  w2_ref, b2_ref, out_ref):
    z = (jnp.dot(p1_ref[...], wa_ref[...], preferred_element_type=jnp.float32)
         + jnp.dot(p2_ref[...], wb_ref[...], preferred_element_type=jnp.float32)
         + jnp.dot(p3_ref[...], wc_ref[...], preferred_element_type=jnp.float32)
         + b1_ref[...])
    z = jnp.maximum(z, 0.0)
    out_ref[...] = (jnp.dot(z, w2_ref[...], preferred_element_type=jnp.float32)
                    + b2_ref[...])


def _full_spec(shape):
    nd = len(shape)
    return pl.BlockSpec(shape, lambda i=0, _n=nd: (0,) * _n)


def _mlp2_call(x, a, w1, b1, w2, b2, g, be, bat3, dh):
    return pl.pallas_call(
        _mlp2_body,
        grid=(_NB,),
        in_specs=[
            pl.BlockSpec((_R, _D), lambda i: (i, 0)),
            pl.BlockSpec((_NC, _R, _D), lambda i: (0, i, 0)),
            _full_spec((_D, dh)),
            _full_spec((1, dh)),
            _full_spec((dh, dh)),
            _full_spec((1, dh)),
            _full_spec((1, dh)),
            _full_spec((1, dh)),
            pl.BlockSpec((1, 1, _R), lambda i: (i, 0, 0)),
        ],
        out_specs=[
            pl.BlockSpec((_R, dh), lambda i: (i, 0)),
            pl.BlockSpec((_G, dh), lambda i: (0, 0)),
        ],
        out_shape=[
            jax.ShapeDtypeStruct((_N, dh), jnp.float32),
            jax.ShapeDtypeStruct((_G, dh), jnp.float32),
        ],
    )(x, a, w1, b1, w2, b2, g, be, bat3)


def _mlp3_call(x, a, w, b, g, be, bat3, dh):
    return pl.pallas_call(
        _mlp3_body,
        grid=(_NB,),
        in_specs=[
            pl.BlockSpec((_R, _D), lambda i: (i, 0)),
            pl.BlockSpec((_NC, _R, _D), lambda i: (0, i, 0)),
            _full_spec((_D, dh)),
            _full_spec((1, dh)),
            _full_spec((1, dh)),
            _full_spec((1, dh)),
            pl.BlockSpec((1, 1, _R), lambda i: (i, 0, 0)),
        ],
        out_specs=pl.BlockSpec((_G, dh), lambda i: (0, 0)),
        out_shape=jax.ShapeDtypeStruct((_G, dh), jnp.float32),
    )(x, a, w, b, g, be, bat3)


def _readout_call(p1, p2, p3, wa, wb, wc, b1, w2, b2):
    return pl.pallas_call(
        _readout_body,
        in_specs=[_full_spec(t.shape) for t in
                  (p1, p2, p3, wa, wb, wc, b1, w2, b2)],
        out_specs=_full_spec((_G, _C)),
        out_shape=jax.ShapeDtypeStruct((_G, _C), jnp.float32),
    )(p1, p2, p3, wa, wb, wc, b1, w2, b2)


def kernel(x, edge_index, batch, W1a, b1a, W1b, b1b, W2a, b2a, W2b, b2b, W3,
           b3, g1, be1, g2, be2, g3, be3, Wl1, bl1, Wl2, bl2):
    src = edge_index[0].reshape(_NSB, _SB, _CH)
    dst = edge_index[1].reshape(_NSB, _SB, _CH)
    bat3 = batch.reshape(_NB, 1, _R)
    zinit = jnp.zeros((_NPAD, _D), jnp.float32)

    r = lambda v: v.reshape(1, -1)

    a = _agg_run(x, src, dst, zinit)
    h1, p1 = _mlp2_call(x, a, W1a, r(b1a), W1b, r(b1b), r(g1), r(be1), bat3,
                        _H)
    a = _agg_run(h1, src, dst, zinit)
    h2, p2 = _mlp2_call(h1, a, W2a, r(b2a), W2b, r(b2b), r(g2), r(be2), bat3,
                        _H)
    a = _agg_run(h2, src, dst, zinit)
    p3 = _mlp3_call(h2, a, W3, r(b3), r(g3), r(be3), bat3, 512)

    return _readout_call(p1, p2, p3, Wl1[:_H], Wl1[_H:2 * _H], Wl1[2 * _H:],
                         r(bl1), Wl2, r(bl2))


# speedup vs baseline: 10.1414x; 1.0214x over previous
import numpy as np
import jax
import jax.numpy as jnp
from jax import lax
from jax.experimental import pallas as pl
from jax.experimental.pallas import tpu as pltpu
from jax.experimental.pallas import tpu_sc as plsc

_N, _E, _D, _H, _G, _C = 10000, 320000, 128, 128, 64, 10
_BN_SCALE = float(1.0 / np.sqrt(1.0 + 1e-5))

_NC, _NS = 2, 16
_NW = _NC * _NS
_CH = 128
_SB = 4
_NSB = _E // (_CH * _SB)
_SPT = _NSB // _NW
_SREM = _NSB - _SPT * _NW
_NPAD = 10112
_RPT = _NPAD // _NS

_R = 2000
_NB = _N // _R


def _agg_body(x_hbm, src_hbm, dst_hbm, zinit_hbm, out_hbm, srcv, dstv, rows_v,
              acc_sh, gsA, gsB, ssA, ssB, isem):
    c = lax.axis_index("c")
    s = lax.axis_index("s")
    w = s * _NC + c

    base = w * _SPT + jnp.minimum(w, _SREM)
    cnt = _SPT + (w < _SREM).astype(jnp.int32)
    gsems = (gsA, gsB)
    ssems = (ssA, ssB)
    dummy = x_hbm.at[pl.ds(0, _CH), :]
    r0 = s * _RPT

    def idx_fire(q, pq):
        pltpu.async_copy(src_hbm.at[q], srcv.at[pq], isem)
        pltpu.async_copy(dst_hbm.at[q], dstv.at[pq], isem)

    def idx_drain(pq):
        pltpu.make_async_copy(dummy, srcv.at[pq], isem).wait()
        pltpu.make_async_copy(dummy, dstv.at[pq], isem).wait()

    def g_fire(pq, j, b):
        pltpu.async_copy(x_hbm.at[srcv.at[pq, j]], rows_v.at[b], gsems[b])

    def s_fire(pq, j, b):
        pltpu.async_copy(rows_v.at[b], acc_sh.at[dstv.at[pq, j]], ssems[b],
                         add=True)

    def g_drain(b):
        pltpu.make_async_copy(dummy, rows_v.at[b], gsems[b]).wait()

    def s_drain(b):
        pltpu.make_async_copy(dummy, rows_v.at[b], ssems[b]).wait()

    idx_fire(base, 0)
    idx_drain(0)
    g_fire(0, 0, 0)
    g_fire(0, 1, 1)
    idx_fire(jnp.minimum(base + 1, _NSB - 1), 1)
    pltpu.sync_copy(zinit_hbm.at[pl.ds(r0, _RPT), :],
                    acc_sh.at[pl.ds(r0, _RPT), :])
    plsc.subcore_barrier()
    g_drain(0); s_fire(0, 0, 0)
    g_drain(1); s_fire(0, 1, 1)
    s_drain(0); g_fire(0, 2, 0)
    s_drain(1); g_fire(0, 3, 1)

    def body(i, carry):
        p = lax.rem(i, 2)
        pp = 1 - p
        idx_drain(p)
        g_drain(0); s_fire(pp, 2, 0)
        g_drain(1); s_fire(pp, 3, 1)
        s_drain(0); g_fire(p, 0, 0)
        s_drain(1); g_fire(p, 1, 1)
        idx_fire(jnp.minimum(base + i + 1, _NSB - 1), pp)
        g_drain(0); s_fire(p, 0, 0)
        g_drain(1); s_fire(p, 1, 1)
        s_drain(0); g_fire(p, 2, 0)
        s_drain(1); g_fire(p, 3, 1)
        return carry

    lax.fori_loop(1, cnt, body, 0)

    pl_ = lax.rem(cnt - 1, 2)
    idx_drain(1 - pl_)
    g_drain(0); s_fire(pl_, 2, 0)
    g_drain(1); s_fire(pl_, 3, 1)
    s_drain(0)
    s_drain(1)

    plsc.subcore_barrier()
    pltpu.sync_copy(acc_sh.at[pl.ds(r0, _RPT), :],
                    out_hbm.at[c, pl.ds(r0, _RPT), :])


_agg_kernel_cache = {}


def _agg_run(x, src, dst, zinit):
    if "k" not in _agg_kernel_cache:
        _agg_kernel_cache["k"] = pl.kernel(
            _agg_body,
            out_type=jax.ShapeDtypeStruct((_NC, _NPAD, _D), jnp.float32),
            mesh=plsc.VectorSubcoreMesh(core_axis_name="c",
                                        subcore_axis_name="s",
                                        num_cores=_NC, num_subcores=_NS),
            scratch_types=[
                pltpu.VMEM((2, _SB, _CH), jnp.int32),
                pltpu.VMEM((2, _SB, _CH), jnp.int32),
                pltpu.VMEM((2, _CH, _D), jnp.float32),
                pltpu.VMEM_SHARED((_NPAD, _D), jnp.float32),
                pltpu.SemaphoreType.DMA,
                pltpu.SemaphoreType.DMA,
                pltpu.SemaphoreType.DMA,
                pltpu.SemaphoreType.DMA,
                pltpu.SemaphoreType.DMA,
            ],
        )
    return _agg_kernel_cache["k"](x, src, dst, zinit)


def _mlp2_body(x_ref, a_ref, w1_ref, b1_ref, w2_ref, b2_ref, g_ref, be_ref,
               bat_ref, h_ref, p_ref):
    i = pl.program_id(0)
    h = x_ref[...] + a_ref[0] + a_ref[1]
    t = jnp.maximum(
        jnp.dot(h, w1_ref[...], preferred_element_type=jnp.float32)
        + b1_ref[...], 0.0)
    u = (jnp.dot(t, w2_ref[...], preferred_element_type=jnp.float32)
         + b2_ref[...])
    hn = jnp.maximum(u * _BN_SCALE * g_ref[...] + be_ref[...], 0.0)
    h_ref[...] = hn
    bat = bat_ref[0, 0, :]
    oh = (bat[:, None] == lax.broadcasted_iota(jnp.int32, (_R, _G), 1)
          ).astype(jnp.float32)
    p = lax.dot_general(oh, hn, (((0,), (0,)), ((), ())),
                        preferred_element_type=jnp.float32)

    @pl.when(i == 0)
    def _():
        p_ref[...] = p

    @pl.when(i != 0)
    def _():
        p_ref[...] = p_ref[...] + p


def _mlp3_body(x_ref, a_ref, w_ref, b_ref, g_ref, be_ref, bat_ref, p_ref):
    i = pl.program_id(0)
    h = x_ref[...] + a_ref[0] + a_ref[1]
    t = jnp.maximum(
        jnp.dot(h, w_ref[...], preferred_element_type=jnp.float32)
        + b_ref[...], 0.0)
    hn = jnp.maximum(t * _BN_SCALE * g_ref[...] + be_ref[...], 0.0)
    bat = bat_ref[0, 0, :]
    oh = (bat[:, None] == lax.broadcasted_iota(jnp.int32, (_R, _G), 1)
          ).astype(jnp.float32)
    p = lax.dot_general(oh, hn, (((0,), (0,)), ((), ())),
                        preferred_element_type=jnp.float32)

    @pl.when(i == 0)
    def _():
        p_ref[...] = p

    @pl.when(i != 0)
    def _():
        p_ref[...] = p_ref[...] + p


def _readout_body(p1_ref, p2_ref, p3_ref, wa_ref, wb_ref, wc_ref, b1_ref,
                  w2_ref, b2_ref, out_ref):
    z = (jnp.dot(p1_ref[...], wa_ref[...], preferred_element_type=jnp.float32)
         + jnp.dot(p2_ref[...], wb_ref[...], preferred_element_type=jnp.float32)
         + jnp.dot(p3_ref[...], wc_ref[...], preferred_element_type=jnp.float32)
         + b1_ref[...])
    z = jnp.maximum(z, 0.0)
    out_ref[...] = (jnp.dot(z, w2_ref[...], preferred_element_type=jnp.float32)
                    + b2_ref[...])


def _full_spec(shape):
    nd = len(shape)
    return pl.BlockSpec(shape, lambda i=0, _n=nd: (0,) * _n)


def _mlp2_call(x, a, w1, b1, w2, b2, g, be, bat3, dh):
    return pl.pallas_call(
        _mlp2_body,
        grid=(_NB,),
        in_specs=[
            pl.BlockSpec((_R, _D), lambda i: (i, 0)),
            pl.BlockSpec((_NC, _R, _D), lambda i: (0, i, 0)),
            _full_spec((_D, dh)),
            _full_spec((1, dh)),
            _full_spec((dh, dh)),
            _full_spec((1, dh)),
            _full_spec((1, dh)),
            _full_spec((1, dh)),
            pl.BlockSpec((1, 1, _R), lambda i: (i, 0, 0)),
        ],
        out_specs=[
            pl.BlockSpec((_R, dh), lambda i: (i, 0)),
            pl.BlockSpec((_G, dh), lambda i: (0, 0)),
        ],
        out_shape=[
            jax.ShapeDtypeStruct((_N, dh), jnp.float32),
            jax.ShapeDtypeStruct((_G, dh), jnp.float32),
        ],
    )(x, a, w1, b1, w2, b2, g, be, bat3)


def _mlp3_call(x, a, w, b, g, be, bat3, dh):
    return pl.pallas_call(
        _mlp3_body,
        grid=(_NB,),
        in_specs=[
            pl.BlockSpec((_R, _D), lambda i: (i, 0)),
            pl.BlockSpec((_NC, _R, _D), lambda i: (0, i, 0)),
            _full_spec((_D, dh)),
            _full_spec((1, dh)),
            _full_spec((1, dh)),
            _full_spec((1, dh)),
            pl.BlockSpec((1, 1, _R), lambda i: (i, 0, 0)),
        ],
        out_specs=pl.BlockSpec((_G, dh), lambda i: (0, 0)),
        out_shape=jax.ShapeDtypeStruct((_G, dh), jnp.float32),
    )(x, a, w, b, g, be, bat3)


def _readout_call(p1, p2, p3, wa, wb, wc, b1, w2, b2):
    return pl.pallas_call(
        _readout_body,
        in_specs=[_full_spec(t.shape) for t in
                  (p1, p2, p3, wa, wb, wc, b1, w2, b2)],
        out_specs=_full_spec((_G, _C)),
        out_shape=jax.ShapeDtypeStruct((_G, _C), jnp.float32),
    )(p1, p2, p3, wa, wb, wc, b1, w2, b2)


def kernel(x, edge_index, batch, W1a, b1a, W1b, b1b, W2a, b2a, W2b, b2b, W3,
           b3, g1, be1, g2, be2, g3, be3, Wl1, bl1, Wl2, bl2):
    src = edge_index[0].reshape(_NSB, _SB, _CH)
    dst = edge_index[1].reshape(_NSB, _SB, _CH)
    bat3 = batch.reshape(_NB, 1, _R)
    zinit = jnp.zeros((_NPAD, _D), jnp.float32)

    r = lambda v: v.reshape(1, -1)

    a = _agg_run(x, src, dst, zinit)
    h1, p1 = _mlp2_call(x, a, W1a, r(b1a), W1b, r(b1b), r(g1), r(be1), bat3,
                        _H)
    a = _agg_run(h1, src, dst, zinit)
    h2, p2 = _mlp2_call(h1, a, W2a, r(b2a), W2b, r(b2b), r(g2), r(be2), bat3,
                        _H)
    a = _agg_run(h2, src, dst, zinit)
    p3 = _mlp3_call(h2, a, W3, r(b3), r(g3), r(be3), bat3, 512)

    return _readout_call(p1, p2, p3, Wl1[:_H], Wl1[_H:2 * _H], Wl1[2 * _H:],
                         r(bl1), Wl2, r(bl2))


# speedup vs baseline: 14.8674x; 1.4660x over previous
import numpy as np
import jax
import jax.numpy as jnp
from jax import lax
from jax.experimental import pallas as pl
from jax.experimental.pallas import tpu as pltpu
from jax.experimental.pallas import tpu_sc as plsc

_N, _E, _D, _H, _G, _C = 10000, 320000, 128, 128, 64, 10
_BN_SCALE = float(1.0 / np.sqrt(1.0 + 1e-5))

_NC, _NS = 2, 16
_NW = _NC * _NS
_CH = 128
_SB = 2
_NSB = _E // (_CH * _SB)
_SPT = _NSB // _NW
_SREM = _NSB - _SPT * _NW
_NGRP = _SPT // 3
_RPT = 632
_LASTR = _N - _RPT * (_NS - 1)

_R = 2000
_NB = _N // _R


def _agg_body(x_hbm, src_hbm, dst_hbm, zinit_hbm, out_hbm, srcv, dstv, rows_v,
              acc_sh, gs0, gs1, gs2, ss0, ss1, ss2, isem):
    c = lax.axis_index("c")
    s = lax.axis_index("s")
    w = s * _NC + c

    base = w * _SPT + jnp.minimum(w, _SREM)
    total = 6 * _NGRP + 2 * (w < _SREM).astype(jnp.int32)
    gsems = (gs0, gs1, gs2)
    ssems = (ss0, ss1, ss2)
    dummy = x_hbm.at[pl.ds(0, _CH), :]
    r0 = s * _RPT

    def idx_fire(q, slot):
        q = jnp.minimum(q, _NSB - 1)
        pltpu.async_copy(src_hbm.at[q], srcv.at[slot], isem)
        pltpu.async_copy(dst_hbm.at[q], dstv.at[slot], isem)

    def idx_drain(slot):
        pltpu.make_async_copy(dummy, srcv.at[slot], isem).wait()
        pltpu.make_async_copy(dummy, dstv.at[slot], isem).wait()

    def g_fire(slot, j, b):
        pltpu.async_copy(x_hbm.at[srcv.at[slot, j]], rows_v.at[b], gsems[b])

    def s_fire(slot, j, b):
        pltpu.async_copy(rows_v.at[b], acc_sh.at[dstv.at[slot, j]], ssems[b],
                         add=True)

    def g_drain(b):
        pltpu.make_async_copy(dummy, rows_v.at[b], gsems[b]).wait()

    def s_drain(b):
        pltpu.make_async_copy(dummy, rows_v.at[b], ssems[b]).wait()

    def zinit_copy():
        @pl.when(s < _NS - 1)
        def _():
            pltpu.sync_copy(zinit_hbm.at[pl.ds(r0, _RPT), :],
                            acc_sh.at[pl.ds(r0, _RPT), :])

        @pl.when(s == _NS - 1)
        def _():
            pltpu.sync_copy(zinit_hbm.at[pl.ds(_RPT * (_NS - 1), _LASTR), :],
                            acc_sh.at[pl.ds(_RPT * (_NS - 1), _LASTR), :])

    def group(g, peel):
        c6 = 6 * g
        if not peel:
            s_drain(2)
        idx_drain(1)
        idx_fire(base + 3 * g + 2, 2)
        g_fire(1, 0, 2)
        g_drain(0); s_fire(0, 0, 0)
        s_drain(0)
        g_fire(1, 1, 0)
        g_drain(1); s_fire(0, 1, 1)
        s_drain(1)
        idx_drain(2)
        idx_fire(base + 3 * g + 3, 0)
        g_fire(2, 0, 1)
        g_drain(2); s_fire(1, 0, 2)
        s_drain(2)
        g_fire(2, 1, 2)
        g_drain(0); s_fire(1, 1, 0)
        s_drain(0)
        idx_drain(0)
        idx_fire(base + 3 * g + 4, 1)

        @pl.when(c6 + 6 < total)
        def _():
            g_fire(0, 0, 0)

        g_drain(1); s_fire(2, 0, 1)
        s_drain(1)

        @pl.when(c6 + 7 < total)
        def _():
            g_fire(0, 1, 1)

        g_drain(2); s_fire(2, 1, 2)

    idx_fire(base, 0)
    idx_drain(0)
    g_fire(0, 0, 0)
    g_fire(0, 1, 1)
    idx_fire(base + 1, 1)
    zinit_copy()
    plsc.subcore_barrier()

    group(0, True)
    lax.fori_loop(1, _NGRP, lambda g, car: (group(g, False), car)[1], 0)

    rem = total > 6 * _NGRP

    @pl.when(rem)
    def _():
        g_drain(0); s_fire(0, 0, 0)
        g_drain(1); s_fire(0, 1, 1)

    s_drain(2)

    @pl.when(rem)
    def _():
        s_drain(0)
        s_drain(1)

    idx_drain(1)

    plsc.subcore_barrier()

    @pl.when(s < _NS - 1)
    def _():
        pltpu.sync_copy(acc_sh.at[pl.ds(r0, _RPT), :],
                        out_hbm.at[c, pl.ds(r0, _RPT), :])

    @pl.when(s == _NS - 1)
    def _():
        pltpu.sync_copy(acc_sh.at[pl.ds(_RPT * (_NS - 1), _LASTR), :],
                        out_hbm.at[c, pl.ds(_RPT * (_NS - 1), _LASTR), :])


_agg_kernel_cache = {}


def _agg_run(x, src, dst, zinit):
    if "k" not in _agg_kernel_cache:
        _agg_kernel_cache["k"] = pl.kernel(
            _agg_body,
            out_type=jax.ShapeDtypeStruct((_NC, _N, _D), jnp.float32),
            mesh=plsc.VectorSubcoreMesh(core_axis_name="c",
                                        subcore_axis_name="s",
                                        num_cores=_NC, num_subcores=_NS),
            scratch_types=[
                pltpu.VMEM((3, _SB, _CH), jnp.int32),
                pltpu.VMEM((3, _SB, _CH), jnp.int32),
                pltpu.VMEM((3, _CH, _D), jnp.float32),
                pltpu.VMEM_SHARED((_N, _D), jnp.float32),
                pltpu.SemaphoreType.DMA,
                pltpu.SemaphoreType.DMA,
                pltpu.SemaphoreType.DMA,
                pltpu.SemaphoreType.DMA,
                pltpu.SemaphoreType.DMA,
                pltpu.SemaphoreType.DMA,
                pltpu.SemaphoreType.DMA,
            ],
        )
    return _agg_kernel_cache["k"](x, src, dst, zinit)


def _mlp2_body(x_ref, a_ref, w1_ref, b1_ref, w2_ref, b2_ref, g_ref, be_ref,
               bat_ref, h_ref, p_ref):
    i = pl.program_id(0)
    h = x_ref[...] + a_ref[0] + a_ref[1]
    t = jnp.maximum(
        jnp.dot(h, w1_ref[...], preferred_element_type=jnp.float32)
        + b1_ref[...], 0.0)
    u = (jnp.dot(t, w2_ref[...], preferred_element_type=jnp.float32)
         + b2_ref[...])
    hn = jnp.maximum(u * _BN_SCALE * g_ref[...] + be_ref[...], 0.0)
    h_ref[...] = hn
    bat = bat_ref[0, 0, :]
    oh = (bat[:, None] == lax.broadcasted_iota(jnp.int32, (_R, _G), 1)
          ).astype(jnp.float32)
    p = lax.dot_general(oh, hn, (((0,), (0,)), ((), ())),
                        preferred_element_type=jnp.float32)

    @pl.when(i == 0)
    def _():
        p_ref[...] = p

    @pl.when(i != 0)
    def _():
        p_ref[...] = p_ref[...] + p


def _mlp3_body(x_ref, a_ref, w_ref, b_ref, g_ref, be_ref, bat_ref, p_ref):
    i = pl.program_id(0)
    h = x_ref[...] + a_ref[0] + a_ref[1]
    t = jnp.maximum(
        jnp.dot(h, w_ref[...], preferred_element_type=jnp.float32)
        + b_ref[...], 0.0)
    hn = jnp.maximum(t * _BN_SCALE * g_ref[...] + be_ref[...], 0.0)
    bat = bat_ref[0, 0, :]
    oh = (bat[:, None] == lax.broadcasted_iota(jnp.int32, (_R, _G), 1)
          ).astype(jnp.float32)
    p = lax.dot_general(oh, hn, (((0,), (0,)), ((), ())),
                        preferred_element_type=jnp.float32)

    @pl.when(i == 0)
    def _():
        p_ref[...] = p

    @pl.when(i != 0)
    def _():
        p_ref[...] = p_ref[...] + p


def _readout_body(p1_ref, p2_ref, p3_ref, wa_ref, wb_ref, wc_ref, b1_ref,
                  w2_ref, b2_ref, out_ref):
    z = (jnp.dot(p1_ref[...], wa_ref[...], preferred_element_type=jnp.float32)
         + jnp.dot(p2_ref[...], wb_ref[...], preferred_element_type=jnp.float32)
         + jnp.dot(p3_ref[...], wc_ref[...], preferred_element_type=jnp.float32)
         + b1_ref[...])
    z = jnp.maximum(z, 0.0)
    out_ref[...] = (jnp.dot(z, w2_ref[...], preferred_element_type=jnp.float32)
                    + b2_ref[...])


def _full_spec(shape):
    nd = len(shape)
    return pl.BlockSpec(shape, lambda i=0, _n=nd: (0,) * _n)


def _mlp2_call(x, a, w1, b1, w2, b2, g, be, bat3, dh):
    return pl.pallas_call(
        _mlp2_body,
        grid=(_NB,),
        in_specs=[
            pl.BlockSpec((_R, _D), lambda i: (i, 0)),
            pl.BlockSpec((_NC, _R, _D), lambda i: (0, i, 0)),
            _full_spec((_D, dh)),
            _full_spec((1, dh)),
            _full_spec((dh, dh)),
            _full_spec((1, dh)),
            _full_spec((1, dh)),
            _full_spec((1, dh)),
            pl.BlockSpec((1, 1, _R), lambda i: (i, 0, 0)),
        ],
        out_specs=[
            pl.BlockSpec((_R, dh), lambda i: (i, 0)),
            pl.BlockSpec((_G, dh), lambda i: (0, 0)),
        ],
        out_shape=[
            jax.ShapeDtypeStruct((_N, dh), jnp.float32),
            jax.ShapeDtypeStruct((_G, dh), jnp.float32),
        ],
    )(x, a, w1, b1, w2, b2, g, be, bat3)


def _mlp3_call(x, a, w, b, g, be, bat3, dh):
    return pl.pallas_call(
        _mlp3_body,
        grid=(_NB,),
        in_specs=[
            pl.BlockSpec((_R, _D), lambda i: (i, 0)),
            pl.BlockSpec((_NC, _R, _D), lambda i: (0, i, 0)),
            _full_spec((_D, dh)),
            _full_spec((1, dh)),
            _full_spec((1, dh)),
            _full_spec((1, dh)),
            pl.BlockSpec((1, 1, _R), lambda i: (i, 0, 0)),
        ],
        out_specs=pl.BlockSpec((_G, dh), lambda i: (0, 0)),
        out_shape=jax.ShapeDtypeStruct((_G, dh), jnp.float32),
    )(x, a, w, b, g, be, bat3)


def _readout_call(p1, p2, p3, wa, wb, wc, b1, w2, b2):
    return pl.pallas_call(
        _readout_body,
        in_specs=[_full_spec(t.shape) for t in
                  (p1, p2, p3, wa, wb, wc, b1, w2, b2)],
        out_specs=_full_spec((_G, _C)),
        out_shape=jax.ShapeDtypeStruct((_G, _C), jnp.float32),
    )(p1, p2, p3, wa, wb, wc, b1, w2, b2)


def kernel(x, edge_index, batch, W1a, b1a, W1b, b1b, W2a, b2a, W2b, b2b, W3,
           b3, g1, be1, g2, be2, g3, be3, Wl1, bl1, Wl2, bl2):
    src = edge_index[0].reshape(_NSB, _SB, _CH)
    dst = edge_index[1].reshape(_NSB, _SB, _CH)
    bat3 = batch.reshape(_NB, 1, _R)
    zinit = jnp.zeros((_N, _D), jnp.float32)

    r = lambda v: v.reshape(1, -1)

    a = _agg_run(x, src, dst, zinit)
    h1, p1 = _mlp2_call(x, a, W1a, r(b1a), W1b, r(b1b), r(g1), r(be1), bat3,
                        _H)
    a = _agg_run(h1, src, dst, zinit)
    h2, p2 = _mlp2_call(h1, a, W2a, r(b2a), W2b, r(b2b), r(g2), r(be2), bat3,
                        _H)
    a = _agg_run(h2, src, dst, zinit)
    p3 = _mlp3_call(h2, a, W3, r(b3), r(g3), r(be3), bat3, 512)

    return _readout_call(p1, p2, p3, Wl1[:_H], Wl1[_H:2 * _H], Wl1[2 * _H:],
                         r(bl1), Wl2, r(bl2))
